# Initial kernel scaffold; baseline (speedup 1.0000x reference)
import functools

import jax
import jax.numpy as jnp
from jax.experimental import pallas as pl
from jax.experimental.pallas import tpu as pltpu

EPS = 1e-6
HI = jax.lax.Precision.HIGHEST
NEG = 0.2


def _dot(a, b):
    return jax.lax.dot_general(a, b, (((1,), (0,)), ((), ())),
                               preferred_element_type=jnp.float32, precision=HI)


def _dot_cc(a, b, ca, cb):
    return jax.lax.dot_general(a, b, (((ca,), (cb,)), ((), ())),
                               preferred_element_type=jnp.float32, precision=HI)


def _slice3(v, n):
    return v[:, 0:n], v[:, n:2 * n], v[:, 2 * n:3 * n]


def _lane_iota(n):
    return jax.lax.broadcasted_iota(jnp.int32, (1, n), 1)


def _norm_of(p):
    p0, p1, p2 = p
    return jnp.sqrt(p0 * p0 + p1 * p1 + p2 * p2 + 1e-12) + EPS


def _vn_nonlin(p, d, mean, inv):
    p0, p1, p2 = p
    d0, d1, d2 = d
    nrm = _norm_of(p)
    nbn = (nrm - mean) * inv
    s = nbn / nrm
    p0, p1, p2 = p0 * s, p1 * s, p2 * s
    dt = p0 * d0 + p1 * d1 + p2 * d2
    d2s = d0 * d0 + d1 * d1 + d2 * d2
    mask = (dt >= 0).astype(jnp.float32)
    q = dt / (d2s + EPS)
    o0 = NEG * p0 + 0.8 * (mask * p0 + (1.0 - mask) * (p0 - q * d0))
    o1 = NEG * p1 + 0.8 * (mask * p1 + (1.0 - mask) * (p1 - q * d1))
    o2 = NEG * p2 + 0.8 * (mask * p2 + (1.0 - mask) * (p2 - q * d2))
    return o0, o1, o2


def _edge_body(x_ref, wf_ref, wd_ref, out_ref, gf_ref, *, N, C, CO, K):
    x = x_ref[...]
    xt = _slice3(x, N)
    G = _dot_cc(xt[0], xt[0], 0, 0)
    G = G + _dot_cc(xt[1], xt[1], 0, 0)
    G = G + _dot_cc(xt[2], xt[2], 0, 0)
    xx = jnp.sum(xt[0] * xt[0] + xt[1] * xt[1] + xt[2] * xt[2],
                 axis=0, keepdims=True)
    sel = 2.0 * G - xx

    wf = wf_ref[...]
    wd = wd_ref[...]
    yf1 = _dot(wf[:, :C], x)
    yf2 = _dot(wf[:, C:], x)
    yd1 = _dot(wd[:, :C], x)
    yd2 = _dot(wd[:, C:], x)
    cf = _slice3(yf2 - yf1, N)
    cd = _slice3(yd2 - yd1, N)
    Af = jnp.concatenate(_slice3(yf1, N), axis=0)
    Ad = jnp.concatenate(_slice3(yd1, N), axis=0)

    lane = _lane_iota(N)
    s1 = jnp.zeros((CO, 1), jnp.float32)
    s2 = jnp.zeros((CO, 1), jnp.float32)
    idxs = []
    for j in range(K):
        idx = jnp.argmax(sel, axis=1, keepdims=True)
        idxs.append(idx)
        hit = idx == lane
        sel = jnp.where(hit, -jnp.inf, sel)
        g = _dot_cc(Af, hit.astype(jnp.float32), 1, 1)
        gf_ref[:, j * N:(j + 1) * N] = g
        p = (g[0:CO] + cf[0], g[CO:2 * CO] + cf[1], g[2 * CO:3 * CO] + cf[2])
        nrm = _norm_of(p)
        s1 = s1 + jnp.sum(nrm, axis=1, keepdims=True)
        s2 = s2 + jnp.sum(nrm * nrm, axis=1, keepdims=True)

    cnt = float(N * K)
    mean = s1 / cnt
    var = s2 / cnt - mean * mean
    inv = jax.lax.rsqrt(var + 1e-5)

    acc0 = jnp.zeros((CO, N), jnp.float32)
    acc1 = jnp.zeros((CO, N), jnp.float32)
    acc2 = jnp.zeros((CO, N), jnp.float32)
    for j in range(K):
        g = gf_ref[:, j * N:(j + 1) * N]
        p = (g[0:CO] + cf[0], g[CO:2 * CO] + cf[1], g[2 * CO:3 * CO] + cf[2])
        oh = (idxs[j] == lane).astype(jnp.float32)
        gd = _dot_cc(Ad, oh, 1, 1)
        d = (gd[0:CO] + cd[0], gd[CO:2 * CO] + cd[1], gd[2 * CO:3 * CO] + cd[2])
        o0, o1, o2 = _vn_nonlin(p, d, mean, inv)
        acc0, acc1, acc2 = acc0 + o0, acc1 + o1, acc2 + o2

    out_ref[:, 0:N] = acc0 / K
    out_ref[:, N:2 * N] = acc1 / K
    out_ref[:, 2 * N:3 * N] = acc2 / K


def _edge(x, wf, wd, N, C, CO, K=20):
    body = functools.partial(_edge_body, N=N, C=C, CO=CO, K=K)
    return pl.pallas_call(
        body,
        out_shape=jax.ShapeDtypeStruct((CO, 3 * N), jnp.float32),
        scratch_shapes=[pltpu.VMEM((3 * CO, K * N), jnp.float32)],
    )(x, wf, wd)


def _fps_body(p_ref, np_ref, *, N, M):
    p0 = p_ref[0:1, :]
    p1 = p_ref[1:2, :]
    p2 = p_ref[2:3, :]
    lane_n = _lane_iota(N)
    lane_m = _lane_iota(M)

    l0, l1, l2 = p0[0, 0], p1[0, 0], p2[0, 0]
    dists = jnp.full((1, N), 1e10, jnp.float32)
    n0 = jnp.where(lane_m == 0, l0, 0.0)
    n1 = jnp.where(lane_m == 0, l1, 0.0)
    n2 = jnp.where(lane_m == 0, l2, 0.0)

    def body(i, st):
        dists, n0, n1, n2, l0, l1, l2 = st
        d = (p0 - l0) ** 2 + (p1 - l1) ** 2 + (p2 - l2) ** 2
        dists = jnp.minimum(dists, d)
        ni = jnp.argmax(dists).astype(jnp.int32)
        hit = lane_n == ni
        l0 = jnp.sum(jnp.where(hit, p0, 0.0))
        l1 = jnp.sum(jnp.where(hit, p1, 0.0))
        l2 = jnp.sum(jnp.where(hit, p2, 0.0))
        n0 = jnp.where(lane_m == i, l0, n0)
        n1 = jnp.where(lane_m == i, l1, n1)
        n2 = jnp.where(lane_m == i, l2, n2)
        return dists, n0, n1, n2, l0, l1, l2

    st = (dists, n0, n1, n2, l0, l1, l2)
    st = jax.lax.fori_loop(1, M, body, st)
    np_ref[0:1, :] = st[1]
    np_ref[1:2, :] = st[2]
    np_ref[2:3, :] = st[3]


def _fps(p, M):
    N = p.shape[1]
    body = functools.partial(_fps_body, N=N, M=M)
    return pl.pallas_call(
        body,
        out_shape=jax.ShapeDtypeStruct((3, M), jnp.float32),
    )(p)


def _td_body(x_ref, p_ref, npt_ref, wf_ref, wd_ref, out_ref, g_ref,
             *, N, M, C, CO, S):
    x = x_ref[...]
    npt = npt_ref[...]
    d0 = npt[:, 0:1] - p_ref[0:1, :]
    d1 = npt[:, 1:2] - p_ref[1:2, :]
    d2 = npt[:, 2:3] - p_ref[2:3, :]
    negd = -(d0 * d0 + d1 * d1 + d2 * d2)

    yf = _dot(wf_ref[...], x)
    yd = _dot(wd_ref[...], x)
    A = jnp.concatenate(_slice3(yf, N) + _slice3(yd, N), axis=0)

    lane = _lane_iota(N)
    s1 = jnp.zeros((CO, 1), jnp.float32)
    s2 = jnp.zeros((CO, 1), jnp.float32)
    for j in range(S):
        idx = jnp.argmax(negd, axis=1, keepdims=True)
        hit = idx == lane
        negd = jnp.where(hit, -jnp.inf, negd)
        g = _dot_cc(A, hit.astype(jnp.float32), 1, 1)
        g_ref[:, j * M:(j + 1) * M] = g
        nrm = _norm_of((g[0:CO], g[CO:2 * CO], g[2 * CO:3 * CO]))
        s1 = s1 + jnp.sum(nrm, axis=1, keepdims=True)
        s2 = s2 + jnp.sum(nrm * nrm, axis=1, keepdims=True)

    cnt = float(M * S)
    mean = s1 / cnt
    var = s2 / cnt - mean * mean
    inv = jax.lax.rsqrt(var + 1e-5)

    acc0 = jnp.zeros((CO, M), jnp.float32)
    acc1 = jnp.zeros((CO, M), jnp.float32)
    acc2 = jnp.zeros((CO, M), jnp.float32)
    for j in range(S):
        g = g_ref[:, j * M:(j + 1) * M]
        p = (g[0:CO], g[CO:2 * CO], g[2 * CO:3 * CO])
        d = (g[3 * CO:4 * CO], g[4 * CO:5 * CO], g[5 * CO:6 * CO])
        o0, o1, o2 = _vn_nonlin(p, d, mean, inv)
        acc0, acc1, acc2 = acc0 + o0, acc1 + o1, acc2 + o2

    out_ref[:, 0:M] = acc0 / S
    out_ref[:, M:2 * M] = acc1 / S
    out_ref[:, 2 * M:3 * M] = acc2 / S


def _td(x, p, npt, wf, wd, N, M, C, CO, S=16):
    body = functools.partial(_td_body, N=N, M=M, C=C, CO=CO, S=S)
    return pl.pallas_call(
        body,
        out_shape=jax.ShapeDtypeStruct((CO, 3 * M), jnp.float32),
        scratch_shapes=[pltpu.VMEM((6 * CO, S * M), jnp.float32)],
    )(x, p, npt, wf, wd)


def _vnl_val(x, wf, wd, N):
    yp = _dot(wf, x)
    yd = _dot(wd, x)
    p = _slice3(yp, N)
    d = _slice3(yd, N)
    nrm = _norm_of(p)
    mean = jnp.mean(nrm, axis=1, keepdims=True)
    dev = nrm - mean
    var = jnp.mean(dev * dev, axis=1, keepdims=True)
    inv = jax.lax.rsqrt(var + 1e-5)
    o0, o1, o2 = _vn_nonlin(p, d, mean, inv)
    return jnp.concatenate([o0, o1, o2], axis=1)


def _vnl_body(x_ref, wf_ref, wd_ref, out_ref, *, N):
    out_ref[...] = _vnl_val(x_ref[...], wf_ref[...], wd_ref[...], N)


def _vnl(x, wf, wd, N):
    body = functools.partial(_vnl_body, N=N)
    return pl.pallas_call(
        body,
        out_shape=jax.ShapeDtypeStruct((wf.shape[0], 3 * N), jnp.float32),
    )(x, wf, wd)


def _tu_body(xs_ref, xd_ref, w1f_ref, w1d_ref, w2f_ref, w2d_ref,
             pdt_ref, ps_ref, out_ref, *, Nd, Ns, CO):
    a = _vnl_val(xs_ref[...], w1f_ref[...], w1d_ref[...], Nd)
    b = _vnl_val(xd_ref[...], w2f_ref[...], w2d_ref[...], Ns)

    pdt = pdt_ref[...]
    d0 = pdt[:, 0:1] - ps_ref[0:1, :]
    d1 = pdt[:, 1:2] - ps_ref[1:2, :]
    d2 = pdt[:, 2:3] - ps_ref[2:3, :]
    negd = -(d0 * d0 + d1 * d1 + d2 * d2)

    lane = _lane_iota(Ns)
    recips = []
    ohs = []
    for _ in range(3):
        mv = jnp.max(negd, axis=1, keepdims=True)
        idx = jnp.argmax(negd, axis=1, keepdims=True)
        hit = idx == lane
        negd = jnp.where(hit, -jnp.inf, negd)
        dist = jnp.sqrt(jnp.maximum(-mv, 1e-12))
        recips.append(1.0 / (dist + 1e-8))
        ohs.append(hit.astype(jnp.float32))
    rsum = recips[0] + recips[1] + recips[2]
    WT = (ohs[0] * (recips[0] / rsum) + ohs[1] * (recips[1] / rsum)
          + ohs[2] * (recips[2] / rsum))

    bt = _slice3(b, Ns)
    at = _slice3(a, Nd)
    for t in range(3):
        interp = _dot_cc(bt[t], WT, 1, 1)
        out_ref[:, t * Nd:(t + 1) * Nd] = at[t] + interp


def _tu(xs, xd, w1f, w1d, w2f, w2d, pdt, ps, Nd, Ns, CO):
    body = functools.partial(_tu_body, Nd=Nd, Ns=Ns, CO=CO)
    return pl.pallas_call(
        body,
        out_shape=jax.ShapeDtypeStruct((CO, 3 * Nd), jnp.float32),
    )(xs, xd, w1f, w1d, w2f, w2d, pdt, ps)


def kernel(x, params):
    P = params
    p1 = jnp.transpose(x[0])
    x0 = p1.reshape(1, 3 * 1024)

    x1 = _edge(x0, P['conv1_Wf'], P['conv1_Wd'], N=1024, C=1, CO=64)

    p2 = _fps(p1, 512)
    x2 = _td(x1, p1, jnp.transpose(p2), P['ds1_Wf'], P['ds1_Wd'],
             N=1024, M=512, C=64, CO=64)
    x2 = _edge(x2, P['conv2_Wf'], P['conv2_Wd'], N=512, C=64, CO=128)

    p3 = _fps(p2, 256)
    x3 = _td(x2, p2, jnp.transpose(p3), P['ds2_Wf'], P['ds2_Wd'],
             N=512, M=256, C=128, CO=128)
    x3 = _edge(x3, P['conv3_Wf'], P['conv3_Wd'], N=256, C=128, CO=256)

    p4 = _fps(p3, 128)
    x4 = _td(x3, p3, jnp.transpose(p4), P['ds3_Wf'], P['ds3_Wd'],
             N=256, M=128, C=256, CO=256)
    x4 = _edge(x4, P['conv4_Wf'], P['conv4_Wd'], N=128, C=256, CO=512)
    x4 = _edge(x4, P['conv5_Wf'], P['conv5_Wd'], N=128, C=512, CO=512)

    x5 = _tu(x3, x4, P['up1m1_Wf'], P['up1m1_Wd'], P['up1m2_Wf'],
             P['up1m2_Wd'], jnp.transpose(p3), p4, Nd=256, Ns=128, CO=256)
    x5 = _edge(x5, P['conv6_Wf'], P['conv6_Wd'], N=256, C=256, CO=256)

    x6 = _tu(x2, x5, P['up2m1_Wf'], P['up2m1_Wd'], P['up2m2_Wf'],
             P['up2m2_Wd'], jnp.transpose(p2), p3, Nd=512, Ns=256, CO=128)
    x6 = _edge(x6, P['conv7_Wf'], P['conv7_Wd'], N=512, C=128, CO=128)

    x7 = _tu(x1, x6, P['up3m1_Wf'], P['up3m1_Wd'], P['up3m2_Wf'],
             P['up3m2_Wd'], jnp.transpose(p1), p2, Nd=1024, Ns=512, CO=64)
    x7 = _edge(x7, P['conv8_Wf'], P['conv8_Wd'], N=1024, C=64, CO=64)

    out = _vnl(x7, P['conv9_Wf'], P['conv9_Wd'], N=1024)
    return out.reshape(1, 64, 3, 1024)


# speedup vs baseline: 4.0271x; 4.0271x over previous
import functools

import jax
import jax.numpy as jnp
from jax.experimental import pallas as pl
from jax.experimental.pallas import tpu as pltpu

EPS = 1e-6
HI = jax.lax.Precision.HIGHEST
DEF = jax.lax.Precision.DEFAULT
NEG = 0.2


def _dot(a, b, prec=DEF):
    return jax.lax.dot_general(a, b, (((1,), (0,)), ((), ())),
                               preferred_element_type=jnp.float32, precision=prec)


def _dot_cc(a, b, ca, cb, prec=HI):
    return jax.lax.dot_general(a, b, (((ca,), (cb,)), ((), ())),
                               preferred_element_type=jnp.float32, precision=prec)


def _slice3(v, n):
    return v[:, 0:n], v[:, n:2 * n], v[:, 2 * n:3 * n]


def _lane_iota(n):
    return jax.lax.broadcasted_iota(jnp.int32, (1, n), 1)


def _norm_of(p):
    p0, p1, p2 = p
    return jnp.sqrt(p0 * p0 + p1 * p1 + p2 * p2 + 1e-12) + EPS


def _vn_nonlin(p, d, mean, sd):
    p0, p1, p2 = p
    d0, d1, d2 = d
    nrm = _norm_of(p)
    nbn = (nrm - mean) / sd
    p0, p1, p2 = p0 / nrm * nbn, p1 / nrm * nbn, p2 / nrm * nbn
    dt = p0 * d0 + p1 * d1 + p2 * d2
    d2s = d0 * d0 + d1 * d1 + d2 * d2
    mask = (dt >= 0).astype(jnp.float32)
    q = dt / (d2s + EPS)
    o0 = NEG * p0 + 0.8 * (mask * p0 + (1.0 - mask) * (p0 - q * d0))
    o1 = NEG * p1 + 0.8 * (mask * p1 + (1.0 - mask) * (p1 - q * d1))
    o2 = NEG * p2 + 0.8 * (mask * p2 + (1.0 - mask) * (p2 - q * d2))
    return o0, o1, o2


def _edge_k1(xc_ref, xf_ref, wf_ref, wd_ref, r_ref, nrm_ref,
             sel_ref, *, N, C, CO, K):
    xf = xf_ref[...]
    G = _dot_cc(xf, xf, 0, 0, DEF)
    inner = -2.0 * G
    xx = jnp.sum(xf * xf, axis=0, keepdims=True)
    xxc = jnp.transpose(xx)
    sel_ref[...] = (-xxc) - inner - xx

    xc = xc_ref[...]
    wcat = jnp.concatenate([wf_ref[...], wd_ref[...]], axis=0)
    CM = 2 * CO
    lane = _lane_iota(N)

    def pass1(j, carry):
        sel = sel_ref[...]
        idx = jnp.argmax(sel, axis=1, keepdims=True)
        hit = idx == lane
        sel_ref[...] = jnp.where(hit, -jnp.inf, sel)
        oh = hit.astype(jnp.float32)
        if 3 * C > 768:
            xg = jnp.concatenate([_dot_cc(xc[0:768], oh, 1, 1, HI),
                                  _dot_cc(xc[768:], oh, 1, 1, HI)], axis=0)
        else:
            xg = _dot_cc(xc, oh, 1, 1, HI)
        for t in range(3):
            ct = xc[t * C:(t + 1) * C]
            edge = jnp.concatenate([xg[t * C:(t + 1) * C] - ct, ct], axis=0)
            r_ref[j, t * CM:(t + 1) * CM] = _dot(wcat, edge)
        p = (r_ref[j, 0:CO], r_ref[j, CM:CM + CO],
             r_ref[j, 2 * CM:2 * CM + CO])
        nrm_ref[j] = _norm_of(p)
        return carry

    jax.lax.fori_loop(0, K, pass1, 0)


def _edge_k2(r_ref, m_ref, v_ref, out_ref, *, N, CO, K):
    CM = 2 * CO
    mean = m_ref[...]
    sd = jnp.sqrt(v_ref[...] + 1e-5)
    out_ref[...] = jnp.zeros((CO, 3 * N), jnp.float32)

    def pass2(j, carry):
        p = (r_ref[j, 0:CO], r_ref[j, CM:CM + CO],
             r_ref[j, 2 * CM:2 * CM + CO])
        d = (r_ref[j, CO:CM], r_ref[j, CM + CO:2 * CM],
             r_ref[j, 2 * CM + CO:3 * CM])
        o0, o1, o2 = _vn_nonlin(p, d, mean, sd)
        out_ref[:, 0:N] += o0
        out_ref[:, N:2 * N] += o1
        out_ref[:, 2 * N:3 * N] += o2
        return carry

    jax.lax.fori_loop(0, K, pass2, 0)
    out_ref[...] = out_ref[...] / K


def _edge_k1e(xc_ref, xf_ref, e_ref, sel_ref, *, N, C, K):
    xf = xf_ref[...]
    G = _dot_cc(xf, xf, 0, 0, DEF)
    inner = -2.0 * G
    xx = jnp.sum(xf * xf, axis=0, keepdims=True)
    sel_ref[...] = (-jnp.transpose(xx)) - inner - xx
    xc = xc_ref[...]
    lane = _lane_iota(N)

    def pass1(j, carry):
        sel = sel_ref[...]
        idx = jnp.argmax(sel, axis=1, keepdims=True)
        hit = idx == lane
        sel_ref[...] = jnp.where(hit, -jnp.inf, sel)
        xg = jnp.concatenate([_dot_cc(xc[0:768], hit.astype(jnp.float32), 1, 1, HI),
                              _dot_cc(xc[768:], hit.astype(jnp.float32), 1, 1, HI)],
                             axis=0)
        for t in range(3):
            ct = xc[t * C:(t + 1) * C]
            e_ref[j, :, t * N:(t + 1) * N] = jnp.concatenate(
                [xg[t * C:(t + 1) * C] - ct, ct], axis=0)
        return carry

    jax.lax.fori_loop(0, K, pass1, 0)


def _edge_k2e(p_ref, d_ref, m_ref, v_ref, out_ref, *, N, CO, K):
    mean = m_ref[...]
    sd = jnp.sqrt(v_ref[...] + 1e-5)
    out_ref[...] = jnp.zeros((CO, 3 * N), jnp.float32)

    def pass2(j, carry):
        p = _slice3(p_ref[j], N)
        d = _slice3(d_ref[j], N)
        o0, o1, o2 = _vn_nonlin(p, d, mean, sd)
        out_ref[:, 0:N] += o0
        out_ref[:, N:2 * N] += o1
        out_ref[:, 2 * N:3 * N] += o2
        return carry

    jax.lax.fori_loop(0, K, pass2, 0)
    out_ref[...] = out_ref[...] / K


def _edge(x, wf, wd, N, C, CO, K=20):
    xf = x.reshape(3 * C, N)
    xc = x.reshape(C, 3, N).transpose(1, 0, 2).reshape(3 * C, N)
    if C == 512:
        k1 = functools.partial(_edge_k1e, N=N, C=C, K=K)
        e = pl.pallas_call(
            k1,
            out_shape=jax.ShapeDtypeStruct((K, 2 * C, 3 * N), jnp.float32),
            scratch_shapes=[pltpu.VMEM((N, N), jnp.float32)],
        )(xc, xf)
        gf5 = jnp.transpose(e.reshape(K, 2 * C, 3, N), (1, 2, 3, 0))[None]
        p5 = jnp.einsum('oi,bi...->bo...', wf, gf5)
        d5 = jnp.einsum('oi,bi...->bo...', wd, gf5)
        n5 = jnp.sqrt(jnp.sum(p5 * p5, axis=2) + 1e-12) + EPS
        axes = (0, 2, 3)
        m = jnp.mean(n5, axis=axes, keepdims=True)[0, :, :, 0]
        v = jnp.var(n5, axis=axes, keepdims=True)[0, :, :, 0]
        pr = jnp.transpose(p5[0], (3, 0, 1, 2)).reshape(K, CO, 3 * N)
        dr = jnp.transpose(d5[0], (3, 0, 1, 2)).reshape(K, CO, 3 * N)
        k2 = functools.partial(_edge_k2e, N=N, CO=CO, K=K)
        return pl.pallas_call(
            k2, out_shape=jax.ShapeDtypeStruct((CO, 3 * N), jnp.float32),
        )(pr, dr, m, v)
    k1 = functools.partial(_edge_k1, N=N, C=C, CO=CO, K=K)
    r, nrm = pl.pallas_call(
        k1,
        out_shape=(jax.ShapeDtypeStruct((K, 6 * CO, N), jnp.float32),
                   jax.ShapeDtypeStruct((K, CO, N), jnp.float32)),
        scratch_shapes=[pltpu.VMEM((N, N), jnp.float32)],
    )(xc, xf, wf, wd)
    n5 = jnp.transpose(nrm, (1, 2, 0))[None]
    axes = (0, 2, 3)
    m = jnp.mean(n5, axis=axes, keepdims=True)[0, :, :, 0]
    v = jnp.var(n5, axis=axes, keepdims=True)[0, :, :, 0]
    k2 = functools.partial(_edge_k2, N=N, CO=CO, K=K)
    return pl.pallas_call(
        k2,
        out_shape=jax.ShapeDtypeStruct((CO, 3 * N), jnp.float32),
    )(r, m, v)


def _fps_body(p_ref, np_ref, *, N, M):
    p0 = p_ref[0:1, :]
    p1 = p_ref[1:2, :]
    p2 = p_ref[2:3, :]
    lane_n = _lane_iota(N)
    lane_m = _lane_iota(M)

    l0, l1, l2 = p0[0, 0], p1[0, 0], p2[0, 0]
    dists = jnp.full((1, N), 1e10, jnp.float32)
    n0 = jnp.where(lane_m == 0, l0, 0.0)
    n1 = jnp.where(lane_m == 0, l1, 0.0)
    n2 = jnp.where(lane_m == 0, l2, 0.0)

    def body(i, st):
        dists, n0, n1, n2, l0, l1, l2 = st
        d = (p0 - l0) ** 2 + (p1 - l1) ** 2 + (p2 - l2) ** 2
        dists = jnp.minimum(dists, d)
        ni = jnp.argmax(dists).astype(jnp.int32)
        hit = lane_n == ni
        l0 = jnp.sum(jnp.where(hit, p0, 0.0))
        l1 = jnp.sum(jnp.where(hit, p1, 0.0))
        l2 = jnp.sum(jnp.where(hit, p2, 0.0))
        n0 = jnp.where(lane_m == i, l0, n0)
        n1 = jnp.where(lane_m == i, l1, n1)
        n2 = jnp.where(lane_m == i, l2, n2)
        return dists, n0, n1, n2, l0, l1, l2

    st = (dists, n0, n1, n2, l0, l1, l2)
    st = jax.lax.fori_loop(1, M, body, st)
    np_ref[0:1, :] = st[1]
    np_ref[1:2, :] = st[2]
    np_ref[2:3, :] = st[3]


def _fps(p, M):
    N = p.shape[1]
    body = functools.partial(_fps_body, N=N, M=M)
    return pl.pallas_call(
        body,
        out_shape=jax.ShapeDtypeStruct((3, M), jnp.float32),
    )(p)


def _td_k1(x_ref, p_ref, npt_ref, wf_ref, wd_ref, g_out, nrm_ref,
           sel_ref, a_ref, *, N, M, C, CO, S):
    x = x_ref[...]
    npt = npt_ref[...]
    d0 = npt[:, 0:1] - p_ref[0:1, :]
    d1 = npt[:, 1:2] - p_ref[1:2, :]
    d2 = npt[:, 2:3] - p_ref[2:3, :]
    sel_ref[...] = -(d0 * d0 + d1 * d1 + d2 * d2)

    yf = _dot(wf_ref[...], x)
    yd = _dot(wd_ref[...], x)
    a_ref[0:3 * CO] = jnp.concatenate(_slice3(yf, N), axis=0)
    a_ref[3 * CO:] = jnp.concatenate(_slice3(yd, N), axis=0)

    lane = _lane_iota(N)

    def pass1(j, carry):
        sel = sel_ref[...]
        idx = jnp.argmax(sel, axis=1, keepdims=True)
        hit = idx == lane
        sel_ref[...] = jnp.where(hit, -jnp.inf, sel)
        g = _dot_cc(a_ref[...], hit.astype(jnp.float32), 1, 1, HI)
        g_out[j] = g
        nrm_ref[j] = _norm_of((g[0:CO], g[CO:2 * CO], g[2 * CO:3 * CO]))
        return carry

    jax.lax.fori_loop(0, S, pass1, 0)


def _td_k2(g_ref, m_ref, v_ref, out_ref, *, M, CO, S):
    mean = m_ref[...]
    sd = jnp.sqrt(v_ref[...] + 1e-5)

    def pass2(j, carry):
        g = g_ref[j]
        p = (g[0:CO], g[CO:2 * CO], g[2 * CO:3 * CO])
        d = (g[3 * CO:4 * CO], g[4 * CO:5 * CO], g[5 * CO:6 * CO])
        o0, o1, o2 = _vn_nonlin(p, d, mean, sd)
        out_ref[j] = jnp.concatenate([o0, o1, o2], axis=1)
        return carry

    jax.lax.fori_loop(0, S, pass2, 0)


def _td(x, p, npt, wf, wd, N, M, C, CO, S=16):
    k1 = functools.partial(_td_k1, N=N, M=M, C=C, CO=CO, S=S)
    g, nrm = pl.pallas_call(
        k1,
        out_shape=(jax.ShapeDtypeStruct((S, 6 * CO, M), jnp.float32),
                   jax.ShapeDtypeStruct((S, CO, M), jnp.float32)),
        scratch_shapes=[pltpu.VMEM((M, N), jnp.float32),
                        pltpu.VMEM((6 * CO, N), jnp.float32)],
    )(x, p, npt, wf, wd)
    n5 = jnp.transpose(nrm, (1, 2, 0))[None]
    axes = (0, 2, 3)
    m = jnp.mean(n5, axis=axes, keepdims=True)[0, :, :, 0]
    v = jnp.var(n5, axis=axes, keepdims=True)[0, :, :, 0]
    k2 = functools.partial(_td_k2, M=M, CO=CO, S=S)
    o = pl.pallas_call(
        k2,
        out_shape=jax.ShapeDtypeStruct((S, CO, 3 * M), jnp.float32),
    )(g, m, v)
    o5 = jnp.transpose(o.reshape(S, CO, 3, M), (1, 2, 3, 0))[None]
    pooled = jnp.mean(o5, axis=-1)
    return pooled[0].reshape(CO, 3 * M)


def _vnl_k1(x_ref, wf_ref, out_ref, *, N):
    yp = _dot(wf_ref[...], x_ref[...])
    out_ref[...] = _norm_of(_slice3(yp, N))


def _vnl_k2(x_ref, wf_ref, wd_ref, m_ref, v_ref, out_ref, *, N):
    yp = _dot(wf_ref[...], x_ref[...])
    yd = _dot(wd_ref[...], x_ref[...])
    mean = m_ref[...]
    sd = jnp.sqrt(v_ref[...] + 1e-5)
    o0, o1, o2 = _vn_nonlin(_slice3(yp, N), _slice3(yd, N), mean, sd)
    out_ref[...] = jnp.concatenate([o0, o1, o2], axis=1)


def _vnl(x, wf, wd, N):
    CO = wf.shape[0]
    k1 = functools.partial(_vnl_k1, N=N)
    n = pl.pallas_call(
        k1, out_shape=jax.ShapeDtypeStruct((CO, N), jnp.float32))(x, wf)
    n3 = n[None]
    m = jnp.mean(n3, axis=(0, 2), keepdims=True)[0]
    v = jnp.var(n3, axis=(0, 2), keepdims=True)[0]
    k2 = functools.partial(_vnl_k2, N=N)
    return pl.pallas_call(
        k2, out_shape=jax.ShapeDtypeStruct((CO, 3 * N), jnp.float32),
    )(x, wf, wd, m, v)


def _tu_body(a_ref, b_ref, pdt_ref, ps_ref, out_ref, *, Nd, Ns, CO):
    a = a_ref[...]
    b = b_ref[...]

    pdt = pdt_ref[...]
    d0 = pdt[:, 0:1] - ps_ref[0:1, :]
    d1 = pdt[:, 1:2] - ps_ref[1:2, :]
    d2 = pdt[:, 2:3] - ps_ref[2:3, :]
    negd = -(d0 * d0 + d1 * d1 + d2 * d2)

    lane = _lane_iota(Ns)
    recips = []
    gs = []
    for _ in range(3):
        mv = jnp.max(negd, axis=1, keepdims=True)
        idx = jnp.argmax(negd, axis=1, keepdims=True)
        hit = idx == lane
        negd = jnp.where(hit, -jnp.inf, negd)
        dist = jnp.sqrt(jnp.maximum(-mv, 1e-12))
        recips.append(1.0 / (dist + 1e-8))
        gs.append(hit.astype(jnp.float32))
    rsum = recips[0] + recips[1] + recips[2]
    w = [jnp.transpose(r / rsum) for r in recips]

    at = _slice3(a, Nd)
    bt = _slice3(b, Ns)
    for t in range(3):
        g0 = _dot_cc(bt[t], gs[0], 1, 1, HI)
        g1 = _dot_cc(bt[t], gs[1], 1, 1, HI)
        g2 = _dot_cc(bt[t], gs[2], 1, 1, HI)
        interp = g0 * w[0] + g1 * w[1] + g2 * w[2]
        out_ref[:, t * Nd:(t + 1) * Nd] = at[t] + interp


def _tu(xs, xd, w1f, w1d, w2f, w2d, pdt, ps, Nd, Ns, CO):
    a = _vnl(xs, w1f, w1d, Nd)
    b = _vnl(xd, w2f, w2d, Ns)
    body = functools.partial(_tu_body, Nd=Nd, Ns=Ns, CO=CO)
    return pl.pallas_call(
        body,
        out_shape=jax.ShapeDtypeStruct((CO, 3 * Nd), jnp.float32),
    )(a, b, pdt, ps)


def kernel(x, params):
    P = params
    p1 = jnp.transpose(x[0])
    x0 = p1.reshape(1, 3 * 1024)

    x1 = _edge(x0, P['conv1_Wf'], P['conv1_Wd'], N=1024, C=1, CO=64)

    p2 = _fps(p1, 512)
    x2 = _td(x1, p1, jnp.transpose(p2), P['ds1_Wf'], P['ds1_Wd'],
             N=1024, M=512, C=64, CO=64)
    x2 = _edge(x2, P['conv2_Wf'], P['conv2_Wd'], N=512, C=64, CO=128)

    p3 = _fps(p2, 256)
    x3 = _td(x2, p2, jnp.transpose(p3), P['ds2_Wf'], P['ds2_Wd'],
             N=512, M=256, C=128, CO=128)
    x3 = _edge(x3, P['conv3_Wf'], P['conv3_Wd'], N=256, C=128, CO=256)

    p4 = _fps(p3, 128)
    x4 = _td(x3, p3, jnp.transpose(p4), P['ds3_Wf'], P['ds3_Wd'],
             N=256, M=128, C=256, CO=256)
    x4 = _edge(x4, P['conv4_Wf'], P['conv4_Wd'], N=128, C=256, CO=512)
    x4 = _edge(x4, P['conv5_Wf'], P['conv5_Wd'], N=128, C=512, CO=512)

    x5 = _tu(x3, x4, P['up1m1_Wf'], P['up1m1_Wd'], P['up1m2_Wf'],
             P['up1m2_Wd'], jnp.transpose(p3), p4, Nd=256, Ns=128, CO=256)
    x5 = _edge(x5, P['conv6_Wf'], P['conv6_Wd'], N=256, C=256, CO=256)

    x6 = _tu(x2, x5, P['up2m1_Wf'], P['up2m1_Wd'], P['up2m2_Wf'],
             P['up2m2_Wd'], jnp.transpose(p2), p3, Nd=512, Ns=256, CO=128)
    x6 = _edge(x6, P['conv7_Wf'], P['conv7_Wd'], N=512, C=128, CO=128)

    x7 = _tu(x1, x6, P['up3m1_Wf'], P['up3m1_Wd'], P['up3m2_Wf'],
             P['up3m2_Wd'], jnp.transpose(p1), p2, Nd=1024, Ns=512, CO=64)
    x7 = _edge(x7, P['conv8_Wf'], P['conv8_Wd'], N=1024, C=64, CO=64)

    out = _vnl(x7, P['conv9_Wf'], P['conv9_Wd'], N=1024)
    return out.reshape(1, 64, 3, 1024)
